# baseline (speedup 1.0000x reference)
import functools

import jax
import jax.numpy as jnp
from jax import lax
from jax.experimental import pallas as pl
from jax.experimental.pallas import tpu as pltpu
from jax.experimental.pallas import tpu_sc as plsc

VOCAB_SZ = 100000
TABLE_SZ = 1000000
NUM_ALT = 4
NTOK = 16384
PSEUDO_CT = 5.0

NWORKERS = 32
CHUNK = NTOK // NWORKERS
NVEC = CHUNK // 16
FLAT = NUM_ALT * CHUNK

_LN2 = 0.6931471805599453
_SQRT2_HALF_BITS = 0x3F3504F3


def _ln(x):
    bits = lax.bitcast_convert_type(x, jnp.int32)
    e = lax.shift_right_arithmetic(bits - _SQRT2_HALF_BITS, 23)
    m = lax.bitcast_convert_type(bits - lax.shift_left(e, 23), jnp.float32)
    s = (m - 1.0) / (m + 1.0)
    z = s * s
    poly = s * (2.0 + z * (2.0 / 3.0 + z * 0.4))
    return e.astype(jnp.float32) * _LN2 + poly


def _mod_table(x):
    q = (x.astype(jnp.float32) * (1.0 / TABLE_SZ)).astype(jnp.int32)
    r = x - q * TABLE_SZ
    r = jnp.where(r < 0, r + TABLE_SZ, r)
    return jnp.where(r >= TABLE_SZ, r - TABLE_SZ, r)


_MESH = plsc.VectorSubcoreMesh(core_axis_name="c", subcore_axis_name="s")


@functools.partial(
    pl.kernel,
    mesh=_MESH,
    out_type=jax.ShapeDtypeStruct((NUM_ALT, NTOK), jnp.float32),
    scratch_types=[
        pltpu.VMEM((CHUNK,), jnp.int32),
        pltpu.VMEM((CHUNK,), jnp.int32),
        pltpu.VMEM((CHUNK,), jnp.int32),
        pltpu.VMEM((FLAT,), jnp.int32),
        pltpu.VMEM((FLAT,), jnp.int32),
        pltpu.VMEM((FLAT,), jnp.int32),
        pltpu.VMEM((FLAT,), jnp.int32),
        pltpu.VMEM((FLAT,), jnp.float32),
        pltpu.VMEM((FLAT,), jnp.float32),
        pltpu.VMEM((FLAT,), jnp.float32),
        pltpu.VMEM((FLAT,), jnp.float32),
        pltpu.VMEM((FLAT,), jnp.float32),
        pltpu.SemaphoreType.DMA,
        pltpu.SemaphoreType.DMA,
        pltpu.SemaphoreType.DMA,
    ],
)
def _score_kernel(tok_hbm, prev_hbm, nxt_hbm,
                  cas0_hbm, cas1_hbm, cas2_hbm, cas3_hbm,
                  uni_hbm, bb_hbm, bf_hbm, tri_hbm, out_hbm,
                  tok_v, prev_v, nxt_v,
                  uidx_v, bbidx_v, bfidx_v, tridx_v,
                  uval_v, bbval_v, bfval_v, trval_v,
                  outb_v, isem, gsem, osem):
    wid = lax.axis_index("s") * 2 + lax.axis_index("c")
    base = wid * CHUNK

    tcp = pltpu.async_copy(tok_hbm.at[pl.ds(base, CHUNK)], tok_v, isem)
    tcp.wait()
    pcp = pltpu.async_copy(prev_hbm.at[pl.ds(base, CHUNK)], prev_v, isem)
    ncp = pltpu.async_copy(nxt_hbm.at[pl.ds(base, CHUNK)], nxt_v, isem)

    ccp = [
        pltpu.async_copy(cas_hbm.at[tok_v],
                         uidx_v.at[pl.ds(a * CHUNK, CHUNK)], isem)
        for a, cas_hbm in enumerate(
            (cas0_hbm, cas1_hbm, cas2_hbm, cas3_hbm))
    ]
    pcp.wait()
    ncp.wait()
    for c in ccp:
        c.wait()

    ucp = pltpu.async_copy(uni_hbm.at[uidx_v], uval_v, gsem)

    def _hash(i, carry):
        for g in range(2):
            off = i * 32 + g * 16
            p = prev_v[pl.ds(off, 16)]
            nx = nxt_v[pl.ds(off, 16)]
            pm = p * 1000003
            n97 = nx * 97
            n31337 = nx * 31337
            for a in range(NUM_ALT):
                fo = a * CHUNK + off
                alt = uidx_v[pl.ds(fo, 16)]
                q = pm + alt * 97
                bbidx_v[pl.ds(fo, 16)] = _mod_table(jnp.abs(q))
                tridx_v[pl.ds(fo, 16)] = _mod_table(jnp.abs(q + n31337))
                bfidx_v[pl.ds(fo, 16)] = (
                    _mod_table(jnp.abs(alt * 1000003 + n97)))
        return carry

    lax.fori_loop(0, NVEC // 2, _hash, 0)

    copies = [
        ucp,
        pltpu.async_copy(bb_hbm.at[bbidx_v], bbval_v, gsem),
        pltpu.async_copy(bf_hbm.at[bfidx_v], bfval_v, gsem),
        pltpu.async_copy(tri_hbm.at[tridx_v], trval_v, gsem),
    ]
    for c in copies:
        c.wait()

    def _score(i, carry):
        for g in range(2):
            off = i * 32 + g * 16
            u, b1, b2, t = [], [], [], []
            for a in range(NUM_ALT):
                fo = a * CHUNK + off
                u.append(uval_v[pl.ds(fo, 16)] + PSEUDO_CT)
                b1.append(bbval_v[pl.ds(fo, 16)] + PSEUDO_CT)
                b2.append(bfval_v[pl.ds(fo, 16)] + PSEUDO_CT)
                t.append(trval_v[pl.ds(fo, 16)] + PSEUDO_CT)
            su = (u[0] + u[1]) + (u[2] + u[3])
            sb1 = (b1[0] + b1[1]) + (b1[2] + b1[3])
            sb2 = (b2[0] + b2[1]) + (b2[2] + b2[3])
            st = (t[0] + t[1]) + (t[2] + t[3])
            den = _ln((su * sb1) * (sb2 * st))
            for a in range(NUM_ALT):
                num = _ln((u[a] * b1[a]) * (b2[a] * t[a]))
                outb_v[pl.ds(a * CHUNK + off, 16)] = num - den
        return carry

    lax.fori_loop(0, NVEC // 2, _score, 0)

    ocp = [
        pltpu.async_copy(outb_v.at[pl.ds(a * CHUNK, CHUNK)],
                         out_hbm.at[a, pl.ds(base, CHUNK)], osem)
        for a in range(NUM_ALT)
    ]
    for c in ocp:
        c.wait()


def kernel(tokens, casing_lookup, uni_counts, bi_back_counts,
           bi_fwd_counts, tri_counts):
    prev = jnp.concatenate([tokens[:1], tokens[:-1]])
    nxt = jnp.concatenate([tokens[1:], tokens[-1:]])
    out = _score_kernel(tokens, prev, nxt,
                        casing_lookup[:, 0], casing_lookup[:, 1],
                        casing_lookup[:, 2], casing_lookup[:, 3],
                        uni_counts, bi_back_counts, bi_fwd_counts,
                        tri_counts)
    return out.T


# speedup vs baseline: 1.0075x; 1.0075x over previous
import functools

import jax
import jax.numpy as jnp
from jax import lax
from jax.experimental import pallas as pl
from jax.experimental.pallas import tpu as pltpu
from jax.experimental.pallas import tpu_sc as plsc

VOCAB_SZ = 100000
TABLE_SZ = 1000000
NUM_ALT = 4
NTOK = 16384
PSEUDO_CT = 5.0

NWORKERS = 32
CHUNK = NTOK // NWORKERS
NVEC = CHUNK // 16
FLAT = NUM_ALT * CHUNK

_LN2 = 0.6931471805599453
_SQRT2_HALF_BITS = 0x3F3504F3


def _ln(x):
    bits = lax.bitcast_convert_type(x, jnp.int32)
    e = lax.shift_right_arithmetic(bits - _SQRT2_HALF_BITS, 23)
    m = lax.bitcast_convert_type(bits - lax.shift_left(e, 23), jnp.float32)
    s = (m - 1.0) / (m + 1.0)
    z = s * s
    poly = s * (2.0 + z * (2.0 / 3.0 + z * 0.4))
    return e.astype(jnp.float32) * _LN2 + poly


def _mod_table(x):
    q = (x.astype(jnp.float32) * (1.0 / TABLE_SZ)).astype(jnp.int32)
    r = x - q * TABLE_SZ
    r = jnp.where(r < 0, r + TABLE_SZ, r)
    return jnp.where(r >= TABLE_SZ, r - TABLE_SZ, r)


_MESH = plsc.VectorSubcoreMesh(core_axis_name="c", subcore_axis_name="s")


@functools.partial(
    pl.kernel,
    mesh=_MESH,
    out_type=jax.ShapeDtypeStruct((NUM_ALT, NTOK), jnp.float32),
    scratch_types=[
        pltpu.VMEM((CHUNK,), jnp.int32),
        pltpu.VMEM((CHUNK,), jnp.int32),
        pltpu.VMEM((CHUNK,), jnp.int32),
        pltpu.VMEM((FLAT,), jnp.int32),
        pltpu.VMEM((FLAT,), jnp.int32),
        pltpu.VMEM((FLAT,), jnp.int32),
        pltpu.VMEM((FLAT,), jnp.int32),
        pltpu.VMEM((FLAT,), jnp.float32),
        pltpu.VMEM((FLAT,), jnp.float32),
        pltpu.VMEM((FLAT,), jnp.float32),
        pltpu.VMEM((FLAT,), jnp.float32),
        pltpu.VMEM((FLAT,), jnp.float32),
        pltpu.SemaphoreType.DMA,
        pltpu.SemaphoreType.DMA,
        pltpu.SemaphoreType.DMA,
    ],
)
def _score_kernel(tok_hbm, prev_hbm, nxt_hbm,
                  cas0_hbm, cas1_hbm, cas2_hbm, cas3_hbm,
                  uni_hbm, bb_hbm, bf_hbm, tri_hbm, out_hbm,
                  tok_v, prev_v, nxt_v,
                  uidx_v, bbidx_v, bfidx_v, tridx_v,
                  uval_v, bbval_v, bfval_v, trval_v,
                  outb_v, isem, gsem, osem):
    wid = lax.axis_index("s") * 2 + lax.axis_index("c")
    base = wid * CHUNK

    tcp = pltpu.async_copy(tok_hbm.at[pl.ds(base, CHUNK)], tok_v, isem)
    tcp.wait()
    pcp = pltpu.async_copy(prev_hbm.at[pl.ds(base, CHUNK)], prev_v, isem)
    ncp = pltpu.async_copy(nxt_hbm.at[pl.ds(base, CHUNK)], nxt_v, isem)

    ccp = [
        pltpu.async_copy(cas_hbm.at[tok_v],
                         uidx_v.at[pl.ds(a * CHUNK, CHUNK)], isem)
        for a, cas_hbm in enumerate(
            (cas0_hbm, cas1_hbm, cas2_hbm, cas3_hbm))
    ]
    pcp.wait()
    ncp.wait()
    for c in ccp:
        c.wait()

    ucp = pltpu.async_copy(uni_hbm.at[uidx_v], uval_v, gsem)

    def _hash(i, carry):
        off = i * 16
        p = prev_v[pl.ds(off, 16)]
        nx = nxt_v[pl.ds(off, 16)]
        pm = p * 1000003
        n97 = nx * 97
        n31337 = nx * 31337
        for a in range(NUM_ALT):
            fo = a * CHUNK + off
            alt = uidx_v[pl.ds(fo, 16)]
            q = pm + alt * 97
            bbidx_v[pl.ds(fo, 16)] = _mod_table(jnp.abs(q))
            tridx_v[pl.ds(fo, 16)] = _mod_table(jnp.abs(q + n31337))
            bfidx_v[pl.ds(fo, 16)] = (
                _mod_table(jnp.abs(alt * 1000003 + n97)))
        return carry

    lax.fori_loop(0, NVEC, _hash, 0)

    copies = [
        ucp,
        pltpu.async_copy(bb_hbm.at[bbidx_v], bbval_v, gsem),
        pltpu.async_copy(bf_hbm.at[bfidx_v], bfval_v, gsem),
        pltpu.async_copy(tri_hbm.at[tridx_v], trval_v, gsem),
    ]
    for c in copies:
        c.wait()

    def _score(i, carry):
        off = i * 16
        u, b1, b2, t = [], [], [], []
        for a in range(NUM_ALT):
            fo = a * CHUNK + off
            u.append(uval_v[pl.ds(fo, 16)] + PSEUDO_CT)
            b1.append(bbval_v[pl.ds(fo, 16)] + PSEUDO_CT)
            b2.append(bfval_v[pl.ds(fo, 16)] + PSEUDO_CT)
            t.append(trval_v[pl.ds(fo, 16)] + PSEUDO_CT)
        su = (u[0] + u[1]) + (u[2] + u[3])
        sb1 = (b1[0] + b1[1]) + (b1[2] + b1[3])
        sb2 = (b2[0] + b2[1]) + (b2[2] + b2[3])
        st = (t[0] + t[1]) + (t[2] + t[3])
        den = _ln((su * sb1) * (sb2 * st))
        for a in range(NUM_ALT):
            num = _ln((u[a] * b1[a]) * (b2[a] * t[a]))
            outb_v[pl.ds(a * CHUNK + off, 16)] = num - den
        return carry

    lax.fori_loop(0, NVEC, _score, 0)

    ocp = [
        pltpu.async_copy(outb_v.at[pl.ds(a * CHUNK, CHUNK)],
                         out_hbm.at[a, pl.ds(base, CHUNK)], osem)
        for a in range(NUM_ALT)
    ]
    for c in ocp:
        c.wait()


def kernel(tokens, casing_lookup, uni_counts, bi_back_counts,
           bi_fwd_counts, tri_counts):
    prev = jnp.concatenate([tokens[:1], tokens[:-1]])
    nxt = jnp.concatenate([tokens[1:], tokens[-1:]])
    out = _score_kernel(tokens, prev, nxt,
                        casing_lookup[:, 0], casing_lookup[:, 1],
                        casing_lookup[:, 2], casing_lookup[:, 3],
                        uni_counts, bi_back_counts, bi_fwd_counts,
                        tri_counts)
    return out.T
